# baseline (speedup 1.0000x reference)
import functools

import jax
import jax.numpy as jnp
from jax import lax
from jax.experimental import pallas as pl
from jax.experimental.pallas import tpu as pltpu
from jax.experimental.pallas import tpu_sc as plsc

_TOPK = 5
_NC = 2
_NS = 16
_NW = _NC * _NS
_L = 16
_BL = 128

_NEG = float(jnp.finfo(jnp.float32).min)


@functools.lru_cache(maxsize=None)
def _build_loss_tc(batch: int, joints: int):
    bg = batch // _BL

    jb = 8
    njb = (joints + jb - 1) // jb

    def body(o_ref, t_ref, w_ref, out_ref):
        d0 = o_ref[:, :, 0, :] - t_ref[:, :, 0, :]
        d1 = o_ref[:, :, 1, :] - t_ref[:, :, 1, :]
        e = d0 * d0 + d1 * d1
        w = w_ref[...].reshape(jb, bg, _BL)
        out_ref[...] = (e * w).reshape(jb * bg, _BL)

    return pl.pallas_call(
        body,
        grid=(njb,),
        in_specs=[
            pl.BlockSpec((jb, bg, 2, _BL), lambda j: (j, 0, 0, 0)),
            pl.BlockSpec((jb, bg, 2, _BL), lambda j: (j, 0, 0, 0)),
            pl.BlockSpec((jb, batch), lambda j: (j, 0)),
        ],
        out_specs=pl.BlockSpec((jb * bg, _BL), lambda j: (j, 0)),
        out_shape=jax.ShapeDtypeStruct((joints * bg, _BL), jnp.float32),
    )


@functools.lru_cache(maxsize=None)
def _build_top5_sc(batch: int, joints: int, interpret: bool = False):
    rows_per_w = batch // _NW
    chunk = 64
    nchunk = rows_per_w // chunk
    ngrp = chunk // _L

    mesh = plsc.VectorSubcoreMesh(
        core_axis_name="c", subcore_axis_name="s", num_cores=_NC,
        num_subcores=_NS)

    @functools.partial(
        pl.kernel,
        out_type=jax.ShapeDtypeStruct((_NW * _L,), jnp.float32),
        mesh=mesh,
        scratch_types=[
            pltpu.VMEM((2, joints, chunk), jnp.float32),
            pltpu.VMEM((_L,), jnp.float32),
            pltpu.SemaphoreType.DMA,
            pltpu.SemaphoreType.DMA,
        ],
        compiler_params=pltpu.CompilerParams(
            use_tc_tiling_on_sc=False, needs_layout_passes=False),
        interpret=interpret,
    )
    def sc_kernel(l_hbm, out_hbm, l_v, acc_v, sem0, sem1):
        cid = lax.axis_index("c")
        sid = lax.axis_index("s")
        wid = sid * _NC + cid
        b0 = wid * rows_per_w
        sems = (sem0, sem1)
        acc_v[...] = jnp.zeros((_L,), jnp.float32)

        def copy(ci, buf):
            b = b0 + ci * chunk
            g = b // _BL
            l0 = b % _BL
            return pltpu.make_async_copy(
                l_hbm.at[:, g, pl.ds(l0, chunk)], l_v.at[buf], sems[buf])

        def process(buf):
            neg = jnp.full((_L,), _NEG, jnp.float32)

            def jbody(j, ms):
                out = []
                for gi in range(ngrp):
                    s = gi * _L
                    m1, m2, m3, m4, m5 = ms[5 * gi:5 * gi + 5]
                    v = l_v[buf, j, pl.ds(s, _L)]
                    n1 = jnp.maximum(m1, v)
                    r = jnp.minimum(m1, v)
                    n2 = jnp.maximum(m2, r)
                    r = jnp.minimum(m2, r)
                    n3 = jnp.maximum(m3, r)
                    r = jnp.minimum(m3, r)
                    n4 = jnp.maximum(m4, r)
                    r = jnp.minimum(m4, r)
                    n5 = jnp.maximum(m5, r)
                    out += [n1, n2, n3, n4, n5]
                return tuple(out)

            ms = lax.fori_loop(0, joints, jbody, (neg,) * (5 * ngrp))
            tot = acc_v[...]
            for gi in range(ngrp):
                m1, m2, m3, m4, m5 = ms[5 * gi:5 * gi + 5]
                tot = tot + (m1 + m2 + m3 + m4 + m5)
            acc_v[...] = tot

        copy(0, 0).start()

        def pipe_body(k, carry):
            ca = 2 * k
            copy(ca + 1, 1).start()
            copy(ca, 0).wait()
            process(0)

            @pl.when(k < (nchunk // 2) - 1)
            def _():
                copy(ca + 2, 0).start()

            copy(ca + 1, 1).wait()
            process(1)
            return carry

        lax.fori_loop(0, nchunk // 2, pipe_body, 0)
        pltpu.sync_copy(acc_v, out_hbm.at[pl.ds(wid * _L, _L)])

    return sc_kernel


def kernel(output, target, target_weight):
    batch, joints, _ = output.shape
    bg = batch // _BL
    o4 = output.reshape(bg, _BL, joints, 2).transpose(2, 0, 3, 1)
    t4 = target.reshape(bg, _BL, joints, 2).transpose(2, 0, 3, 1)
    wt = target_weight.T
    loss = _build_loss_tc(batch, joints)(o4, t4, wt)
    loss3 = loss.reshape(joints, bg, _BL)
    parts = _build_top5_sc(batch, joints)(loss3)
    return jnp.sum(parts) * (1.0 / (_TOPK * batch))


# speedup vs baseline: 1.1504x; 1.1504x over previous
import functools

import jax
import jax.numpy as jnp
from jax import lax
from jax.experimental import pallas as pl
from jax.experimental.pallas import tpu as pltpu
from jax.experimental.pallas import tpu_sc as plsc

_TOPK = 5
_NC = 2
_NS = 16
_NW = _NC * _NS
_L = 16
_BL = 128

_NEG = float(jnp.finfo(jnp.float32).min)


@functools.lru_cache(maxsize=None)
def _build(batch: int, joints: int, interpret: bool = False):
    rows_per_w = batch // _NW
    chunk = 64
    nchunk = rows_per_w // chunk
    ngrp = chunk // _L

    mesh = plsc.VectorSubcoreMesh(
        core_axis_name="c", subcore_axis_name="s", num_cores=_NC,
        num_subcores=_NS)

    @functools.partial(
        pl.kernel,
        out_type=jax.ShapeDtypeStruct((_NW * _L,), jnp.float32),
        mesh=mesh,
        scratch_types=[
            pltpu.VMEM((2, 2, joints, chunk), jnp.float32),
            pltpu.VMEM((2, 2, joints, chunk), jnp.float32),
            pltpu.VMEM((2, joints, chunk), jnp.float32),
            pltpu.VMEM((_L,), jnp.float32),
            pltpu.SemaphoreType.DMA,
            pltpu.SemaphoreType.DMA,
        ],
        compiler_params=pltpu.CompilerParams(
            use_tc_tiling_on_sc=False, needs_layout_passes=False),
        interpret=interpret,
    )
    def sc_kernel(o_hbm, t_hbm, w_hbm, out_hbm, o_v, t_v, w_v, acc_v,
                  sem0, sem1):
        cid = lax.axis_index("c")
        sid = lax.axis_index("s")
        wid = sid * _NC + cid
        b0 = wid * rows_per_w
        sems = (sem0, sem1)
        acc_v[...] = jnp.zeros((_L,), jnp.float32)

        def copies(ci, buf):
            b = b0 + ci * chunk
            g = b // _BL
            l0 = b % _BL
            return (
                pltpu.make_async_copy(
                    o_hbm.at[:, g, 0, pl.ds(l0, chunk)], o_v.at[buf, 0],
                    sems[buf]),
                pltpu.make_async_copy(
                    o_hbm.at[:, g, 1, pl.ds(l0, chunk)], o_v.at[buf, 1],
                    sems[buf]),
                pltpu.make_async_copy(
                    t_hbm.at[:, g, 0, pl.ds(l0, chunk)], t_v.at[buf, 0],
                    sems[buf]),
                pltpu.make_async_copy(
                    t_hbm.at[:, g, 1, pl.ds(l0, chunk)], t_v.at[buf, 1],
                    sems[buf]),
                pltpu.make_async_copy(
                    w_hbm.at[:, pl.ds(b, chunk)], w_v.at[buf], sems[buf]),
            )

        def start(ci, buf):
            for c in copies(ci, buf):
                c.start()

        def wait(ci, buf):
            for c in copies(ci, buf):
                c.wait()

        def process(buf):
            neg = jnp.full((_L,), _NEG, jnp.float32)

            def jbody(j, ms):
                out = []
                for gi in range(ngrp):
                    s = gi * _L
                    m1, m2, m3, m4, m5 = ms[5 * gi:5 * gi + 5]
                    o0 = o_v[buf, 0, j, pl.ds(s, _L)]
                    o1 = o_v[buf, 1, j, pl.ds(s, _L)]
                    t0 = t_v[buf, 0, j, pl.ds(s, _L)]
                    t1 = t_v[buf, 1, j, pl.ds(s, _L)]
                    tw = w_v[buf, j, pl.ds(s, _L)]
                    d0 = o0 - t0
                    d1 = o1 - t1
                    v = (d0 * d0 + d1 * d1) * tw
                    n1 = jnp.maximum(m1, v)
                    r = jnp.minimum(m1, v)
                    n2 = jnp.maximum(m2, r)
                    r = jnp.minimum(m2, r)
                    n3 = jnp.maximum(m3, r)
                    r = jnp.minimum(m3, r)
                    n4 = jnp.maximum(m4, r)
                    r = jnp.minimum(m4, r)
                    n5 = jnp.maximum(m5, r)
                    out += [n1, n2, n3, n4, n5]
                return tuple(out)

            ms = lax.fori_loop(0, joints, jbody, (neg,) * (5 * ngrp))
            tot = acc_v[...]
            for gi in range(ngrp):
                m1, m2, m3, m4, m5 = ms[5 * gi:5 * gi + 5]
                tot = tot + (m1 + m2 + m3 + m4 + m5)
            acc_v[...] = tot

        start(0, 0)

        def pipe_body(k, carry):
            ca = 2 * k
            start(ca + 1, 1)
            wait(ca, 0)
            process(0)

            @pl.when(k < (nchunk // 2) - 1)
            def _():
                start(ca + 2, 0)

            wait(ca + 1, 1)
            process(1)
            return carry

        lax.fori_loop(0, nchunk // 2, pipe_body, 0)
        pltpu.sync_copy(acc_v, out_hbm.at[pl.ds(wid * _L, _L)])

    return sc_kernel


def kernel(output, target, target_weight):
    batch, joints, _ = output.shape
    bg = batch // _BL
    o4 = output.reshape(bg, _BL, joints, 2).transpose(2, 0, 3, 1)
    t4 = target.reshape(bg, _BL, joints, 2).transpose(2, 0, 3, 1)
    wt = target_weight.T
    parts = _build(batch, joints)(o4, t4, wt)
    return jnp.sum(parts) * (1.0 / (_TOPK * batch))


# speedup vs baseline: 1.1533x; 1.0026x over previous
import functools

import jax
import jax.numpy as jnp
from jax import lax
from jax.experimental import pallas as pl
from jax.experimental.pallas import tpu as pltpu
from jax.experimental.pallas import tpu_sc as plsc

_TOPK = 5
_NC = 2
_NS = 16
_NW = _NC * _NS
_L = 16
_BL = 128

_NEG = float(jnp.finfo(jnp.float32).min)


@functools.lru_cache(maxsize=None)
def _build(batch: int, joints: int, interpret: bool = False):
    rows_per_w = batch // _NW
    chunk = 64
    nchunk = rows_per_w // chunk
    ngrp = chunk // _L

    mesh = plsc.VectorSubcoreMesh(
        core_axis_name="c", subcore_axis_name="s", num_cores=_NC,
        num_subcores=_NS)

    @functools.partial(
        pl.kernel,
        out_type=jax.ShapeDtypeStruct((_NW * _L,), jnp.float32),
        mesh=mesh,
        scratch_types=[
            pltpu.VMEM((2, 2, joints, chunk), jnp.float32),
            pltpu.VMEM((2, 2, joints, chunk), jnp.float32),
            pltpu.VMEM((2, joints, chunk), jnp.float32),
            pltpu.VMEM((_L,), jnp.float32),
            pltpu.SemaphoreType.DMA,
            pltpu.SemaphoreType.DMA,
        ],
        compiler_params=pltpu.CompilerParams(
            use_tc_tiling_on_sc=False, needs_layout_passes=False),
        interpret=interpret,
    )
    def sc_kernel(o_hbm, t_hbm, w_hbm, out_hbm, o_v, t_v, w_v, acc_v,
                  sem0, sem1):
        cid = lax.axis_index("c")
        sid = lax.axis_index("s")
        wid = cid * _NS + sid
        b0 = wid * rows_per_w
        sems = (sem0, sem1)
        acc_v[...] = jnp.zeros((_L,), jnp.float32)

        def copies(ci, buf):
            b = b0 + ci * chunk
            g = b // _BL
            l0 = b % _BL
            return (
                pltpu.make_async_copy(
                    o_hbm.at[:, g, 0, pl.ds(l0, chunk)], o_v.at[buf, 0],
                    sems[buf]),
                pltpu.make_async_copy(
                    o_hbm.at[:, g, 1, pl.ds(l0, chunk)], o_v.at[buf, 1],
                    sems[buf]),
                pltpu.make_async_copy(
                    t_hbm.at[:, g, 0, pl.ds(l0, chunk)], t_v.at[buf, 0],
                    sems[buf]),
                pltpu.make_async_copy(
                    t_hbm.at[:, g, 1, pl.ds(l0, chunk)], t_v.at[buf, 1],
                    sems[buf]),
                pltpu.make_async_copy(
                    w_hbm.at[:, pl.ds(b, chunk)], w_v.at[buf], sems[buf]),
            )

        def start(ci, buf):
            for c in copies(ci, buf):
                c.start()

        def wait(ci, buf):
            for c in copies(ci, buf):
                c.wait()

        def process(buf):
            neg = jnp.full((_L,), _NEG, jnp.float32)

            def jbody(j, ms):
                out = []
                for gi in range(ngrp):
                    s = gi * _L
                    m1, m2, m3, m4, m5 = ms[5 * gi:5 * gi + 5]
                    o0 = o_v[buf, 0, j, pl.ds(s, _L)]
                    o1 = o_v[buf, 1, j, pl.ds(s, _L)]
                    t0 = t_v[buf, 0, j, pl.ds(s, _L)]
                    t1 = t_v[buf, 1, j, pl.ds(s, _L)]
                    tw = w_v[buf, j, pl.ds(s, _L)]
                    d0 = o0 - t0
                    d1 = o1 - t1
                    v = (d0 * d0 + d1 * d1) * tw
                    n1 = jnp.maximum(m1, v)
                    r = jnp.minimum(m1, v)
                    n2 = jnp.maximum(m2, r)
                    r = jnp.minimum(m2, r)
                    n3 = jnp.maximum(m3, r)
                    r = jnp.minimum(m3, r)
                    n4 = jnp.maximum(m4, r)
                    r = jnp.minimum(m4, r)
                    n5 = jnp.maximum(m5, r)
                    out += [n1, n2, n3, n4, n5]
                return tuple(out)

            ms = lax.fori_loop(0, joints, jbody, (neg,) * (5 * ngrp))
            tot = acc_v[...]
            for gi in range(ngrp):
                m1, m2, m3, m4, m5 = ms[5 * gi:5 * gi + 5]
                tot = tot + (m1 + m2 + m3 + m4 + m5)
            acc_v[...] = tot

        start(0, 0)

        def pipe_body(k, carry):
            ca = 2 * k
            start(ca + 1, 1)
            wait(ca, 0)
            process(0)

            @pl.when(k < (nchunk // 2) - 1)
            def _():
                start(ca + 2, 0)

            wait(ca + 1, 1)
            process(1)
            return carry

        lax.fori_loop(0, nchunk // 2, pipe_body, 0)
        pltpu.sync_copy(acc_v, out_hbm.at[pl.ds(wid * _L, _L)])

    return sc_kernel


def kernel(output, target, target_weight):
    batch, joints, _ = output.shape
    bg = batch // _BL
    o4 = output.reshape(bg, _BL, joints, 2).transpose(2, 0, 3, 1)
    t4 = target.reshape(bg, _BL, joints, 2).transpose(2, 0, 3, 1)
    wt = target_weight.T
    parts = _build(batch, joints)(o4, t4, wt)
    return jnp.sum(parts) * (1.0 / (_TOPK * batch))
